# baseline (speedup 1.0000x reference)
import functools
import math

import jax
import jax.numpy as jnp
from jax import lax
from jax.experimental import pallas as pl
from jax.experimental.pallas import tpu as pltpu
from jax.experimental.pallas import tpu_sc as plsc

N_SLOTS_MOD = 200
_EVENT_LOGIT_THRESH = math.log(0.85 / 0.15)

_BB = 64
_SCHUNK = 40


def _tc_body(x_ref, slots_ref, ptr_ref, wv_ref, bv_ref, we_ref, be_ref,
             pe_ref, sw_ref, wp_ref, bp_ref,
             copy_ref, hmem_ref, src_ref, nptr_ref, didx_ref):
    i = pl.program_id(0)
    bb, s, d = slots_ref.shape

    x = x_ref[...]
    e_logit = jnp.sum(x * we_ref[...], axis=1, keepdims=True) + be_ref[...]
    evt = e_logit > _EVENT_LOGIT_THRESH
    evt_f = evt.astype(jnp.float32)
    v = lax.dot_general(x, wv_ref[...], (((1,), (1,)), ((), ())),
                        preferred_element_type=jnp.float32) + bv_ref[...]

    swv = sw_ref[...]
    swe = jnp.exp(swv - jnp.max(swv))
    w = swe / jnp.sum(swe)

    ptrv = ptr_ref[...]
    ptr3 = ptrv[:, :, None]

    base = jnp.zeros((bb, d), jnp.float32)
    old = jnp.zeros((bb, d), jnp.float32)
    w_at_ptr = jnp.zeros((bb, 1), jnp.float32)
    for j in range(s // _SCHUNK):
        sl = slice(j * _SCHUNK, (j + 1) * _SCHUNK)
        c = slots_ref[:, sl, :]
        copy_ref[:, sl, :] = c
        selc = (j * _SCHUNK
                + lax.broadcasted_iota(jnp.int32, (bb, _SCHUNK, 1), 1)
                == ptr3).astype(jnp.float32)
        wc = w[None, sl, :]
        base = base + jnp.sum(c * wc, axis=1)
        old = old + jnp.sum(c * selc, axis=1)
        w_at_ptr = w_at_ptr + jnp.sum(wc * selc, axis=1)

    pos_c = jnp.sum(pe_ref[...] * w, axis=0, keepdims=True)
    fused = base + pos_c + evt_f * w_at_ptr * (v - old)

    hmem_ref[...] = lax.dot_general(fused, wp_ref[...], (((1,), (1,)), ((), ())),
                                    preferred_element_type=jnp.float32) + bp_ref[...]

    src_ref[...] = jnp.where(evt, v, old)
    nptr_ref[...] = lax.rem(ptrv + evt.astype(jnp.int32), N_SLOTS_MOD)
    b_glob = i * bb + lax.broadcasted_iota(jnp.int32, (bb, 1), 0)
    didx_ref[...] = b_glob * s + ptrv


_SC_CORES = 2
_SC_SUBCORES = 16


def _make_sc_scatter(total_rows, d, n_workers):
    rows_per_w = total_rows // n_workers
    mesh = plsc.VectorSubcoreMesh(core_axis_name="c", subcore_axis_name="s",
                                  num_cores=_SC_CORES, num_subcores=_SC_SUBCORES)

    @functools.partial(
        pl.kernel,
        mesh=mesh,
        scratch_types=[
            pltpu.VMEM((rows_per_w,), jnp.int32),
            pltpu.VMEM((rows_per_w, d), jnp.float32),
            pltpu.SemaphoreType.DMA,
        ],
    )
    def sc_scatter(ns_ref, src_hbm, idx_hbm, idx_v, rows_v, sem):
        wid = lax.axis_index("s") * _SC_CORES + lax.axis_index("c")
        pltpu.sync_copy(idx_hbm.at[wid], idx_v)
        pltpu.sync_copy(src_hbm.at[pl.ds(wid * rows_per_w, rows_per_w)], rows_v)
        pltpu.async_copy(rows_v, ns_ref.at[idx_v], sem).wait()

    return sc_scatter


def kernel(x_t, slots, ptr, W_value, b_value, W_event, b_event, pos_emb,
           slot_weights, W_proj, b_proj):
    B, S, D = slots.shape
    H = W_proj.shape[0]
    nb = B // _BB

    ptr2 = ptr.reshape(B, 1).astype(jnp.int32)

    grid_spec = pl.GridSpec(
        grid=(nb,),
        in_specs=[
            pl.BlockSpec((_BB, D), lambda i: (i, 0)),
            pl.BlockSpec((_BB, S, D), lambda i: (i, 0, 0)),
            pl.BlockSpec((_BB, 1), lambda i: (i, 0)),
            pl.BlockSpec((D, D), lambda i: (0, 0)),
            pl.BlockSpec((1, D), lambda i: (0, 0)),
            pl.BlockSpec((1, D), lambda i: (0, 0)),
            pl.BlockSpec((1, 1), lambda i: (0, 0)),
            pl.BlockSpec((S, D), lambda i: (0, 0)),
            pl.BlockSpec((S, 1), lambda i: (0, 0)),
            pl.BlockSpec((H, D), lambda i: (0, 0)),
            pl.BlockSpec((1, H), lambda i: (0, 0)),
        ],
        out_specs=[
            pl.BlockSpec((_BB, S, D), lambda i: (i, 0, 0)),
            pl.BlockSpec((_BB, H), lambda i: (i, 0)),
            pl.BlockSpec((_BB, D), lambda i: (i, 0)),
            pl.BlockSpec((_BB, 1), lambda i: (i, 0)),
            pl.BlockSpec((_BB, 1), lambda i: (i, 0)),
        ],
    )
    copy, h_mem, src_rows, nptr2, didx2 = pl.pallas_call(
        _tc_body,
        grid_spec=grid_spec,
        out_shape=[
            jax.ShapeDtypeStruct((B, S, D), jnp.float32),
            jax.ShapeDtypeStruct((B, H), jnp.float32),
            jax.ShapeDtypeStruct((B, D), jnp.float32),
            jax.ShapeDtypeStruct((B, 1), jnp.int32),
            jax.ShapeDtypeStruct((B, 1), jnp.int32),
        ],
        compiler_params=pltpu.CompilerParams(
            dimension_semantics=("arbitrary",),
        ),
    )(x_t, slots, ptr2, W_value, b_value.reshape(1, D), W_event,
      b_event.reshape(1, 1), pos_emb, slot_weights.reshape(S, 1), W_proj,
      b_proj.reshape(1, H))

    nw = _SC_CORES * _SC_SUBCORES
    ns_ref = jax.new_ref(copy.reshape(B * S, D))
    _make_sc_scatter(B, D, nw)(ns_ref, src_rows, didx2.reshape(nw, B // nw))
    new_slots = ns_ref[...].reshape(B, S, D)

    return (h_mem, new_slots, nptr2.reshape(B).astype(ptr.dtype))


# speedup vs baseline: 1.0650x; 1.0650x over previous
import functools
import math

import jax
import jax.numpy as jnp
from jax import lax
from jax.experimental import pallas as pl
from jax.experimental.pallas import tpu as pltpu
from jax.experimental.pallas import tpu_sc as plsc

N_SLOTS_MOD = 200
_EVENT_LOGIT_THRESH = math.log(0.85 / 0.15)

_BB = 64


def _tc_body(x_ref, slots_ref, ptr_ref, wv_ref, bv_ref, we_ref, be_ref,
             pe_ref, sw_ref, wp_ref, bp_ref,
             copy_ref, hmem_ref, src_ref, nptr_ref, didx_ref):
    i = pl.program_id(0)
    bb, s, d = slots_ref.shape

    blk = slots_ref[...]
    copy_ref[...] = blk

    x = x_ref[...]
    e_logit = jnp.sum(x * we_ref[...], axis=1, keepdims=True) + be_ref[...]
    evt = e_logit > _EVENT_LOGIT_THRESH
    evt_f = evt.astype(jnp.float32)
    v = lax.dot_general(x, wv_ref[...], (((1,), (1,)), ((), ())),
                        preferred_element_type=jnp.float32) + bv_ref[...]

    swv = sw_ref[...]
    swe = jnp.exp(swv - jnp.max(swv))
    w = swe / jnp.sum(swe)

    ptrv = ptr_ref[...]
    sel = (lax.broadcasted_iota(jnp.int32, (bb, s, 1), 1)
           == ptrv[:, :, None]).astype(jnp.float32)

    old = jnp.sum(blk * sel, axis=1)
    w_at_ptr = jnp.sum(w[None] * sel, axis=1)
    base = jnp.sum(blk * w[None], axis=1)
    pos_c = jnp.sum(pe_ref[...] * w, axis=0, keepdims=True)
    fused = base + pos_c + evt_f * w_at_ptr * (v - old)

    hmem_ref[...] = lax.dot_general(fused, wp_ref[...], (((1,), (1,)), ((), ())),
                                    preferred_element_type=jnp.float32) + bp_ref[...]

    src_ref[...] = jnp.where(evt, v, old)
    nptr_ref[...] = lax.rem(ptrv + evt.astype(jnp.int32), N_SLOTS_MOD)
    b_glob = i * bb + lax.broadcasted_iota(jnp.int32, (bb, 1), 0)
    didx_ref[...] = b_glob * s + ptrv


_SC_CORES = 2
_SC_SUBCORES = 16


def _make_sc_scatter(total_rows, d, n_workers):
    rows_per_w = total_rows // n_workers
    mesh = plsc.VectorSubcoreMesh(core_axis_name="c", subcore_axis_name="s",
                                  num_cores=_SC_CORES, num_subcores=_SC_SUBCORES)

    @functools.partial(
        pl.kernel,
        mesh=mesh,
        scratch_types=[
            pltpu.VMEM((rows_per_w,), jnp.int32),
            pltpu.VMEM((rows_per_w, d), jnp.float32),
            pltpu.SemaphoreType.DMA,
        ],
    )
    def sc_scatter(ns_ref, src_hbm, idx_hbm, idx_v, rows_v, sem):
        wid = lax.axis_index("s") * _SC_CORES + lax.axis_index("c")
        pltpu.sync_copy(idx_hbm.at[wid], idx_v)
        pltpu.sync_copy(src_hbm.at[pl.ds(wid * rows_per_w, rows_per_w)], rows_v)
        pltpu.async_copy(rows_v, ns_ref.at[idx_v], sem).wait()

    return sc_scatter


def kernel(x_t, slots, ptr, W_value, b_value, W_event, b_event, pos_emb,
           slot_weights, W_proj, b_proj):
    B, S, D = slots.shape
    H = W_proj.shape[0]
    nb = B // _BB

    ptr2 = ptr.reshape(B, 1).astype(jnp.int32)

    grid_spec = pl.GridSpec(
        grid=(nb,),
        in_specs=[
            pl.BlockSpec((_BB, D), lambda i: (i, 0)),
            pl.BlockSpec((_BB, S, D), lambda i: (i, 0, 0)),
            pl.BlockSpec((_BB, 1), lambda i: (i, 0)),
            pl.BlockSpec((D, D), lambda i: (0, 0)),
            pl.BlockSpec((1, D), lambda i: (0, 0)),
            pl.BlockSpec((1, D), lambda i: (0, 0)),
            pl.BlockSpec((1, 1), lambda i: (0, 0)),
            pl.BlockSpec((S, D), lambda i: (0, 0)),
            pl.BlockSpec((S, 1), lambda i: (0, 0)),
            pl.BlockSpec((H, D), lambda i: (0, 0)),
            pl.BlockSpec((1, H), lambda i: (0, 0)),
        ],
        out_specs=[
            pl.BlockSpec((_BB, S, D), lambda i: (i, 0, 0)),
            pl.BlockSpec((_BB, H), lambda i: (i, 0)),
            pl.BlockSpec((_BB, D), lambda i: (i, 0)),
            pl.BlockSpec((_BB, 1), lambda i: (i, 0)),
            pl.BlockSpec((_BB, 1), lambda i: (i, 0)),
        ],
    )
    copy, h_mem, src_rows, nptr2, didx2 = pl.pallas_call(
        _tc_body,
        grid_spec=grid_spec,
        out_shape=[
            jax.ShapeDtypeStruct((B, S, D), jnp.float32),
            jax.ShapeDtypeStruct((B, H), jnp.float32),
            jax.ShapeDtypeStruct((B, D), jnp.float32),
            jax.ShapeDtypeStruct((B, 1), jnp.int32),
            jax.ShapeDtypeStruct((B, 1), jnp.int32),
        ],
        compiler_params=pltpu.CompilerParams(
            dimension_semantics=("parallel",),
        ),
    )(x_t, slots, ptr2, W_value, b_value.reshape(1, D), W_event,
      b_event.reshape(1, 1), pos_emb, slot_weights.reshape(S, 1), W_proj,
      b_proj.reshape(1, H))

    nw = _SC_CORES * _SC_SUBCORES
    ns_ref = jax.new_ref(copy.reshape(B * S, D))
    _make_sc_scatter(B, D, nw)(ns_ref, src_rows, didx2.reshape(nw, B // nw))
    new_slots = ns_ref[...].reshape(B, S, D)

    return (h_mem, new_slots, nptr2.reshape(B).astype(ptr.dtype))
